# baseline (speedup 1.0000x reference)
import functools

import jax
import jax.numpy as jnp
from jax import lax
from jax.experimental import pallas as pl
from jax.experimental.pallas import tpu as pltpu
from jax.experimental.pallas import tpu_sc as plsc


def _gather_fn(n_rows, d, chunk, n_workers):
    rows_per_w = n_rows // n_workers
    n_chunks = rows_per_w // chunk
    mesh = plsc.VectorSubcoreMesh(core_axis_name="c", subcore_axis_name="s")

    @functools.partial(
        pl.kernel,
        mesh=mesh,
        out_type=jax.ShapeDtypeStruct((n_rows, d), jnp.float32),
        compiler_params=pltpu.CompilerParams(use_tc_tiling_on_sc=False),
        scratch_types=[
            pltpu.VMEM((chunk,), jnp.int32),
            pltpu.VMEM((chunk, d), jnp.float32),
            pltpu.SemaphoreType.DMA,
        ],
    )
    def k(idx_hbm, table_hbm, out_hbm, idx_v, rows_v, sem):
        wid = lax.axis_index("s") * 2 + lax.axis_index("c")
        base = wid * rows_per_w

        def body(g, carry):
            off = base + g * chunk
            pltpu.sync_copy(idx_hbm.at[pl.ds(off, chunk)], idx_v)
            pltpu.async_copy(table_hbm.at[idx_v], rows_v, sem).wait()
            pltpu.sync_copy(rows_v, out_hbm.at[pl.ds(off, chunk)])
            return carry

        lax.fori_loop(0, n_chunks, body, 0)

    return k


def kernel(input_ids, table):
    b, s = input_ids.shape
    v, d = table.shape
    n = b * s
    idx = input_ids.reshape(n).astype(jnp.int32)
    out = _gather_fn(n, d, 1024, 32)(idx, table)
    return out.reshape(b, s, d)


# speedup vs baseline: 1.0152x; 1.0152x over previous
import functools

import jax
import jax.numpy as jnp
from jax import lax
from jax.experimental import pallas as pl
from jax.experimental.pallas import tpu as pltpu
from jax.experimental.pallas import tpu_sc as plsc

_NBUF = 2


def _gather_fn(n_rows, d, chunk, n_workers):
    rows_per_w = n_rows // n_workers
    n_chunks = rows_per_w // chunk
    assert n_chunks % _NBUF == 0 and n_chunks // _NBUF >= 2
    mesh = plsc.VectorSubcoreMesh(core_axis_name="c", subcore_axis_name="s")

    @functools.partial(
        pl.kernel,
        mesh=mesh,
        out_type=jax.ShapeDtypeStruct((n_rows, d), jnp.float32),
        compiler_params=pltpu.CompilerParams(use_tc_tiling_on_sc=False),
        scratch_types=[
            pltpu.VMEM((rows_per_w,), jnp.int32),
            *[pltpu.VMEM((chunk, d), jnp.float32) for _ in range(_NBUF)],
            *[pltpu.SemaphoreType.DMA for _ in range(2 * _NBUF)],
        ],
    )
    def k(idx_hbm, table_hbm, out_hbm, idx_v, *bufs_and_sems):
        rows_v = bufs_and_sems[:_NBUF]
        gsem = bufs_and_sems[_NBUF:2 * _NBUF]
        osem = bufs_and_sems[2 * _NBUF:]
        wid = lax.axis_index("s") * 2 + lax.axis_index("c")
        base = wid * rows_per_w

        def idx_slice(g):
            return idx_v.at[pl.ds(g * chunk, chunk)]

        def start_gather(g, b):
            pltpu.async_copy(table_hbm.at[idx_slice(g)], rows_v[b], gsem[b])

        def wait_gather(g, b):
            pltpu.make_async_copy(table_hbm.at[idx_slice(g)], rows_v[b],
                                  gsem[b]).wait()

        def out_slice(g):
            return out_hbm.at[pl.ds(base + g * chunk, chunk)]

        pltpu.sync_copy(idx_hbm.at[pl.ds(base, rows_per_w)], idx_v)

        for b in range(_NBUF):
            start_gather(b, b)

        def body(i, carry):
            for b in range(_NBUF):
                g = i * _NBUF + b
                wait_gather(g, b)
                pltpu.async_copy(rows_v[b], out_slice(g), osem[b])

                @pl.when(i * _NBUF + b + _NBUF < n_chunks)
                def _():
                    pltpu.make_async_copy(rows_v[b], out_slice(g),
                                          osem[b]).wait()
                    start_gather(g + _NBUF, b)

            return carry

        lax.fori_loop(0, n_chunks // _NBUF, body, 0)

        for b in range(_NBUF):
            g = n_chunks - _NBUF + b
            pltpu.make_async_copy(rows_v[b], out_slice(g), osem[b]).wait()

    return k


def kernel(input_ids, table):
    b, s = input_ids.shape
    v, d = table.shape
    n = b * s
    idx = input_ids.reshape(n).astype(jnp.int32)
    out = _gather_fn(n, d, 800, 32)(idx, table)
    return out.reshape(b, s, d)


# speedup vs baseline: 1.3489x; 1.3287x over previous
import functools

import jax
import jax.numpy as jnp
from jax import lax
from jax.experimental import pallas as pl
from jax.experimental.pallas import tpu as pltpu
from jax.experimental.pallas import tpu_sc as plsc

_NBUF = 4


def _gather_fn(n_batch, seq, d, n_workers):
    w = 2 * d
    rows_per_w = n_batch // n_workers
    idx_per_w = rows_per_w * seq
    n_chunks = rows_per_w
    assert n_chunks % _NBUF == 0 and n_chunks // _NBUF >= 2
    mesh = plsc.VectorSubcoreMesh(core_axis_name="c", subcore_axis_name="s")

    @functools.partial(
        pl.kernel,
        mesh=mesh,
        out_type=jax.ShapeDtypeStruct((n_batch, seq, w), jnp.float32),
        compiler_params=pltpu.CompilerParams(use_tc_tiling_on_sc=False),
        scratch_types=[
            pltpu.VMEM((idx_per_w,), jnp.int32),
            *[pltpu.VMEM((seq, d), jnp.float32) for _ in range(_NBUF)],
            *[pltpu.SemaphoreType.DMA for _ in range(2 * _NBUF)],
        ],
    )
    def k(idx_hbm, table_hbm, out_hbm, idx_v, *bufs_and_sems):
        rows_v = bufs_and_sems[:_NBUF]
        gsem = bufs_and_sems[_NBUF:2 * _NBUF]
        osem = bufs_and_sems[2 * _NBUF:]
        wid = lax.axis_index("s") * 2 + lax.axis_index("c")
        base = wid * idx_per_w
        row0 = wid * rows_per_w

        def idx_slice(g):
            return idx_v.at[pl.ds(g * seq, seq)]

        def start_gather(g, b):
            pltpu.async_copy(table_hbm.at[idx_slice(g)], rows_v[b], gsem[b])

        def wait_gather(g, b):
            pltpu.make_async_copy(table_hbm.at[idx_slice(g)], rows_v[b],
                                  gsem[b]).wait()

        def out_slice(g):
            return out_hbm.at[row0 + g, :, pl.ds(0, d)]

        pltpu.sync_copy(idx_hbm.at[pl.ds(base, idx_per_w)], idx_v)

        for b in range(_NBUF):
            start_gather(b, b)

        def body(i, carry):
            for b in range(_NBUF):
                g = i * _NBUF + b
                wait_gather(g, b)
                pltpu.async_copy(rows_v[b], out_slice(g), osem[b])

                @pl.when(i * _NBUF + b + _NBUF < n_chunks)
                def _():
                    pltpu.make_async_copy(rows_v[b], out_slice(g),
                                          osem[b]).wait()
                    start_gather(g + _NBUF, b)

            return carry

        lax.fori_loop(0, n_chunks // _NBUF, body, 0)

        for b in range(_NBUF):
            g = n_chunks - _NBUF + b
            pltpu.make_async_copy(rows_v[b], out_slice(g), osem[b]).wait()

    return k


def kernel(input_ids, table):
    b, s = input_ids.shape
    v, d = table.shape
    idx = input_ids.reshape(b * s).astype(jnp.int32)
    wide = _gather_fn(b, s, d, 32)(idx, table)
    return wide[:, :, :d]
